# Initial kernel scaffold; baseline (speedup 1.0000x reference)
import jax
import jax.numpy as jnp
from jax import lax
from jax.experimental import pallas as pl
from jax.experimental.pallas import tpu as pltpu
from jax.experimental.pallas import tpu_sc as plsc

N = 10000
E = 320000
D = 128
H = 8
DH = 16
DFF = 512
CLAMP = 5.0

NC = 2
NS = 16
NW = NC * NS
E_PER_W = E // NW
CHUNK = 80
NCHUNK = E_PER_W // CHUNK
ROW = 144
N_PER_T = N // NS
ZROWS = 125


def _dot_t(x, w):
    return lax.dot_general(x, w, (((1,), (1,)), ((), ())),
                           preferred_element_type=jnp.float32)


def _proj_body(feat_ref, wq_ref, wk_ref, wv_ref, q_ref, kv_ref):
    x = feat_ref[...]
    q_ref[...] = _dot_t(x, wq_ref[...])
    kv_ref[:, :D] = _dot_t(x, wk_ref[...])
    kv_ref[:, D:] = _dot_t(x, wv_ref[...])


def _proj(feat, wq, wk, wv):
    blk = 1000
    grid = N // blk
    return pl.pallas_call(
        _proj_body,
        grid=(grid,),
        in_specs=[
            pl.BlockSpec((blk, D), lambda i: (i, 0)),
            pl.BlockSpec((D, D), lambda i: (0, 0)),
            pl.BlockSpec((D, D), lambda i: (0, 0)),
            pl.BlockSpec((D, D), lambda i: (0, 0)),
        ],
        out_specs=[
            pl.BlockSpec((blk, D), lambda i: (i, 0)),
            pl.BlockSpec((blk, 2 * D), lambda i: (i, 0)),
        ],
        out_shape=[
            jax.ShapeDtypeStruct((N, D), jnp.float32),
            jax.ShapeDtypeStruct((N, 2 * D), jnp.float32),
        ],
    )(feat, wq, wk, wv)


def _edge_sc(q_hbm, kv_hbm, edge_hbm, out_hbm,
             sidx, didx, qbuf, kvbuf, obuf, zbuf, table, sem):
    c = lax.axis_index("c")
    s = lax.axis_index("s")
    wid = c * NS + s

    lane = lax.iota(jnp.int32, 16)
    zeros = jnp.zeros((16,), jnp.float32)

    def _zb(i, _):
        for j in range(ROW // 16):
            zbuf[i, pl.ds(j * 16, 16)] = zeros
        return 0
    lax.fori_loop(0, ZROWS, _zb, 0)
    for k in range(N_PER_T // ZROWS):
        pltpu.sync_copy(zbuf, table.at[pl.ds(s * N_PER_T + k * ZROWS, ZROWS)])
    plsc.subcore_barrier()

    base = wid * E_PER_W

    def _chunk(g, _):
        start = base + g * CHUNK
        pltpu.sync_copy(edge_hbm.at[0, pl.ds(start, CHUNK)], sidx)
        pltpu.sync_copy(edge_hbm.at[1, pl.ds(start, CHUNK)], didx)
        cp_q = pltpu.async_copy(q_hbm.at[didx], qbuf, sem)
        cp_kv = pltpu.async_copy(kv_hbm.at[sidx], kvbuf, sem)
        cp_q.wait()
        cp_kv.wait()

        def _edge_one(e, _):
            srow = zeros
            for h in range(H):
                qv = qbuf[e, pl.ds(h * DH, DH)]
                kv = kvbuf[e, pl.ds(h * DH, DH)]
                cs = plsc.cumsum(qv * kv)
                u = cs[15] * 0.25
                u = jnp.minimum(jnp.maximum(u, -CLAMP), CLAMP)
                sv = jnp.exp(jnp.full((16,), u, jnp.float32))
                vv = kvbuf[e, pl.ds(D + h * DH, DH)]
                obuf[e, pl.ds(h * DH, DH)] = sv * vv
                srow = jnp.where(lane == h, sv, srow)
            obuf[e, pl.ds(D, 16)] = srow
            return 0
        lax.fori_loop(0, CHUNK, _edge_one, 0)

        pltpu.sync_copy(obuf, table.at[didx], add=True)
        return 0
    lax.fori_loop(0, NCHUNK, _chunk, 0)

    plsc.subcore_barrier()
    pltpu.sync_copy(table.at[pl.ds(s * N_PER_T, N_PER_T)],
                    out_hbm.at[c, pl.ds(s * N_PER_T, N_PER_T)])


def _edge(q, kv, edge_index):
    mesh = plsc.VectorSubcoreMesh(core_axis_name="c", subcore_axis_name="s",
                                  num_cores=NC, num_subcores=NS)
    f = pl.kernel(
        _edge_sc,
        out_type=jax.ShapeDtypeStruct((NC, N, ROW), jnp.float32),
        mesh=mesh,
        scratch_types=[
            pltpu.VMEM((CHUNK,), jnp.int32),
            pltpu.VMEM((CHUNK,), jnp.int32),
            pltpu.VMEM((CHUNK, D), jnp.float32),
            pltpu.VMEM((CHUNK, 2 * D), jnp.float32),
            pltpu.VMEM((CHUNK, ROW), jnp.float32),
            pltpu.VMEM((ZROWS, ROW), jnp.float32),
            pltpu.VMEM_SHARED((N, ROW), jnp.float32),
            pltpu.SemaphoreType.DMA,
        ],
    )
    return f(q, kv, edge_index)


def _layernorm(x, g, b):
    m = jnp.mean(x, axis=-1, keepdims=True)
    xc = x - m
    v = jnp.mean(xc * xc, axis=-1, keepdims=True)
    return xc * lax.rsqrt(v + 1e-5) * g + b


def _post_body(p_ref, feat_ref, wo_ref, exp_ref, g1_ref, b1_ref,
               w1_ref, bb1_ref, w2_ref, bb2_ref, g2_ref, b2_ref, out_ref):
    t = p_ref[0] + p_ref[1]
    sv = t[:, :D]
    dn = t[:, D:D + H]
    dn = jnp.where(dn > 0.0, dn, 1.0)
    rexp = lax.dot_general(1.0 / dn, exp_ref[...], (((1,), (0,)), ((), ())),
                           preferred_element_type=jnp.float32)
    av = sv * rexp
    uh = _dot_t(av, wo_ref[...])
    h1 = _layernorm(feat_ref[...] + uh, g1_ref[...], b1_ref[...])
    hid = jnp.maximum(_dot_t(h1, w1_ref[...]) + bb1_ref[...], 0.0)
    ffn = _dot_t(hid, w2_ref[...]) + bb2_ref[...]
    out_ref[...] = _layernorm(h1 + ffn, g2_ref[...], b2_ref[...])


def _post(parts, feat, wo, expander, g1, b1, w1, bb1, w2, bb2, g2, b2):
    blk = 1000
    grid = N // blk

    def full(shape):
        return pl.BlockSpec(shape, lambda i: tuple(0 for _ in shape))

    return pl.pallas_call(
        _post_body,
        grid=(grid,),
        in_specs=[
            pl.BlockSpec((NC, blk, ROW), lambda i: (0, i, 0)),
            pl.BlockSpec((blk, D), lambda i: (i, 0)),
            full((D, D)),
            full((H, D)),
            full((1, D)),
            full((1, D)),
            full((DFF, D)),
            full((1, DFF)),
            full((D, DFF)),
            full((1, D)),
            full((1, D)),
            full((1, D)),
        ],
        out_specs=pl.BlockSpec((blk, D), lambda i: (i, 0)),
        out_shape=jax.ShapeDtypeStruct((N, D), jnp.float32),
    )(parts, feat, wo, expander, g1, b1, w1, bb1, w2, bb2, g2, b2)


@jax.jit
def kernel(feat, edge_index, Wq, Wk, Wv, Wo, ln1_g, ln1_b,
           w1, b1, w2, b2, ln2_g, ln2_b):
    q, kv = _proj(feat, Wq, Wk, Wv)
    parts = _edge(q, kv, edge_index)
    expander = jnp.repeat(jnp.eye(H, dtype=jnp.float32), DH, axis=1)
    return _post(parts, feat, Wo, expander,
                 ln1_g.reshape(1, D), ln1_b.reshape(1, D),
                 w1, b1.reshape(1, DFF), w2, b2.reshape(1, D),
                 ln2_g.reshape(1, D), ln2_b.reshape(1, D))


# speedup vs baseline: 1.6648x; 1.6648x over previous
import jax
import jax.numpy as jnp
from jax import lax
from jax.experimental import pallas as pl
from jax.experimental.pallas import tpu as pltpu
from jax.experimental.pallas import tpu_sc as plsc

N = 10000
E = 320000
D = 128
H = 8
DH = 16
DFF = 512
CLAMP = 5.0

NC = 2
NS = 16
NW = NC * NS
E_PER_W = E // NW
CHUNK = 80
NCHUNK = E_PER_W // CHUNK
N_TAB = 10240
N_PER_T = N_TAB // NS


def _dot_t(x, w):
    return lax.dot_general(x, w, (((1,), (1,)), ((), ())),
                           preferred_element_type=jnp.float32)


def _proj_body(feat_ref, wq_ref, wk_ref, wv_ref, q_ref, kv_ref):
    x = feat_ref[...]
    q_ref[...] = _dot_t(x, wq_ref[...])
    kv_ref[:, :D] = _dot_t(x, wk_ref[...])
    kv_ref[:, D:] = _dot_t(x, wv_ref[...])


def _proj(feat, wq, wk, wv):
    blk = 1000
    grid = N // blk
    return pl.pallas_call(
        _proj_body,
        grid=(grid,),
        in_specs=[
            pl.BlockSpec((blk, D), lambda i: (i, 0)),
            pl.BlockSpec((D, D), lambda i: (0, 0)),
            pl.BlockSpec((D, D), lambda i: (0, 0)),
            pl.BlockSpec((D, D), lambda i: (0, 0)),
        ],
        out_specs=[
            pl.BlockSpec((blk, D), lambda i: (i, 0)),
            pl.BlockSpec((blk, 2 * D), lambda i: (i, 0)),
        ],
        out_shape=[
            jax.ShapeDtypeStruct((N, D), jnp.float32),
            jax.ShapeDtypeStruct((N, 2 * D), jnp.float32),
        ],
    )(feat, wq, wk, wv)


def _edge_sc(q_hbm, kv_hbm, src_hbm, dst_hbm, sv_hbm, s_hbm,
             sidx, didx, qbuf, kvbuf, obuf, sbuf, table, sem):
    c = lax.axis_index("c")
    s = lax.axis_index("s")
    wid = c * NS + s

    lane = lax.iota(jnp.int32, 16)
    zeros = jnp.zeros((16,), jnp.float32)

    def _zb(i, _):
        for j in range(D // 16):
            obuf[i, pl.ds(j * 16, 16)] = zeros
        return 0
    lax.fori_loop(0, CHUNK, _zb, 0)
    for k in range(N_PER_T // CHUNK):
        pltpu.sync_copy(obuf, table.at[pl.ds(s * N_PER_T + k * CHUNK, CHUNK)])
    plsc.subcore_barrier()

    base = wid * E_PER_W

    def _chunk(g, _):
        start = base + g * CHUNK
        pltpu.sync_copy(src_hbm.at[pl.ds(start, CHUNK)], sidx)
        pltpu.sync_copy(dst_hbm.at[pl.ds(start, CHUNK)], didx)
        cp_q = pltpu.async_copy(q_hbm.at[didx], qbuf, sem)
        cp_kv = pltpu.async_copy(kv_hbm.at[sidx], kvbuf, sem)
        cp_q.wait()
        cp_kv.wait()

        def _edge_one(e, _):
            srow = zeros
            for h in range(H):
                qv = qbuf[e, pl.ds(h * DH, DH)]
                kv = kvbuf[e, pl.ds(h * DH, DH)]
                cs = plsc.cumsum(qv * kv)
                u = cs[15] * 0.25
                u = jnp.minimum(jnp.maximum(u, -CLAMP), CLAMP)
                sv = jnp.exp(jnp.full((16,), u, jnp.float32))
                vv = kvbuf[e, pl.ds(D + h * DH, DH)]
                obuf[e, pl.ds(h * DH, DH)] = sv * vv
                srow = jnp.where(lane == h, sv, srow)
            sbuf[pl.ds(e * H, 16)] = srow
            return 0
        lax.fori_loop(0, CHUNK, _edge_one, 0)

        pltpu.sync_copy(obuf, table.at[didx], add=True)
        pltpu.sync_copy(sbuf.at[pl.ds(0, CHUNK * H)],
                        s_hbm.at[pl.ds(start * H, CHUNK * H)])
        return 0
    lax.fori_loop(0, NCHUNK, _chunk, 0)

    plsc.subcore_barrier()
    pltpu.sync_copy(table.at[pl.ds(s * N_PER_T, N_PER_T)],
                    sv_hbm.at[c, pl.ds(s * N_PER_T, N_PER_T)])


def _edge(q, kv, src, dst):
    mesh = plsc.VectorSubcoreMesh(core_axis_name="c", subcore_axis_name="s",
                                  num_cores=NC, num_subcores=NS)
    f = pl.kernel(
        _edge_sc,
        out_type=(
            jax.ShapeDtypeStruct((NC, N_TAB, D), jnp.float32),
            jax.ShapeDtypeStruct((E * H,), jnp.float32),
        ),
        mesh=mesh,
        compiler_params=pltpu.CompilerParams(needs_layout_passes=False),
        scratch_types=[
            pltpu.VMEM((CHUNK,), jnp.int32),
            pltpu.VMEM((CHUNK,), jnp.int32),
            pltpu.VMEM((CHUNK, D), jnp.float32),
            pltpu.VMEM((CHUNK, 2 * D), jnp.float32),
            pltpu.VMEM((CHUNK, D), jnp.float32),
            pltpu.VMEM((CHUNK * H + 16,), jnp.float32),
            pltpu.VMEM_SHARED((N_TAB, D), jnp.float32),
            pltpu.SemaphoreType.DMA,
        ],
    )
    return f(q, kv, src, dst)


def _den_sc(dst_hbm, s_hbm, dn_hbm, didx2, sbuf, dnbuf):
    c = lax.axis_index("c")
    s = lax.axis_index("s")
    wid = c * NS + s

    lane = lax.iota(jnp.int32, 16)
    lane8 = lane < H
    zeros = jnp.zeros((16,), jnp.float32)

    def _zd(i, _):
        for j in range(4):
            dnbuf[pl.ds(i * 64 + j * 16, 16)] = zeros
        return 0
    lax.fori_loop(0, N * H // 64, _zd, 0)

    base = wid * E_PER_W

    def _chunk(g, _):
        start = base + g * CHUNK
        pltpu.sync_copy(dst_hbm.at[pl.ds(start, CHUNK)],
                        didx2.at[pl.ds(0, CHUNK)])
        pltpu.sync_copy(s_hbm.at[pl.ds(start * H, CHUNK * H)],
                        sbuf.at[pl.ds(0, CHUNK * H)])

        def _edge_one(e, _):
            dvec = didx2[pl.ds(e, 16)]
            didxv = jnp.full((16,), dvec[0] * H, jnp.int32) + lane
            svec = sbuf[pl.ds(e * H, 16)]
            plsc.addupdate_scatter(dnbuf, [didxv], svec, mask=lane8)
            return 0
        lax.fori_loop(0, CHUNK, _edge_one, 0)
        return 0
    lax.fori_loop(0, NCHUNK, _chunk, 0)

    pltpu.sync_copy(dnbuf, dn_hbm.at[wid])


def _den(dst, s_flat):
    mesh = plsc.VectorSubcoreMesh(core_axis_name="c", subcore_axis_name="s",
                                  num_cores=NC, num_subcores=NS)
    f = pl.kernel(
        _den_sc,
        out_type=jax.ShapeDtypeStruct((NW, N * H), jnp.float32),
        mesh=mesh,
        compiler_params=pltpu.CompilerParams(needs_layout_passes=False),
        scratch_types=[
            pltpu.VMEM((CHUNK + 16,), jnp.int32),
            pltpu.VMEM((CHUNK * H + 16,), jnp.float32),
            pltpu.VMEM((N * H,), jnp.float32),
        ],
    )
    return f(dst, s_flat)


def _layernorm(x, g, b):
    m = jnp.mean(x, axis=-1, keepdims=True)
    xc = x - m
    v = jnp.mean(xc * xc, axis=-1, keepdims=True)
    return xc * lax.rsqrt(v + 1e-5) * g + b


def _post_body(p_ref, dn_ref, feat_ref, wo_ref, exp_ref, g1_ref, b1_ref,
               w1_ref, bb1_ref, w2_ref, bb2_ref, g2_ref, b2_ref, out_ref):
    sv = p_ref[0] + p_ref[1]
    dn = jnp.sum(dn_ref[...], axis=0)
    dn = jnp.where(dn > 0.0, dn, 1.0)
    rexp = lax.dot_general(1.0 / dn, exp_ref[...], (((1,), (0,)), ((), ())),
                           preferred_element_type=jnp.float32)
    av = sv * rexp
    uh = _dot_t(av, wo_ref[...])
    h1 = _layernorm(feat_ref[...] + uh, g1_ref[...], b1_ref[...])
    hid = jnp.maximum(_dot_t(h1, w1_ref[...]) + bb1_ref[...], 0.0)
    ffn = _dot_t(hid, w2_ref[...]) + bb2_ref[...]
    out_ref[...] = _layernorm(h1 + ffn, g2_ref[...], b2_ref[...])


def _post(parts, dparts, feat, wo, expander, g1, b1, w1, bb1, w2, bb2, g2, b2):
    blk = 1000
    grid = N // blk

    def full(shape):
        return pl.BlockSpec(shape, lambda i: tuple(0 for _ in shape))

    return pl.pallas_call(
        _post_body,
        grid=(grid,),
        in_specs=[
            pl.BlockSpec((NC, blk, D), lambda i: (0, i, 0)),
            pl.BlockSpec((NW, blk, H), lambda i: (0, i, 0)),
            pl.BlockSpec((blk, D), lambda i: (i, 0)),
            full((D, D)),
            full((H, D)),
            full((1, D)),
            full((1, D)),
            full((DFF, D)),
            full((1, DFF)),
            full((D, DFF)),
            full((1, D)),
            full((1, D)),
            full((1, D)),
        ],
        out_specs=pl.BlockSpec((blk, D), lambda i: (i, 0)),
        out_shape=jax.ShapeDtypeStruct((N, D), jnp.float32),
    )(parts, dparts, feat, wo, expander, g1, b1, w1, bb1, w2, bb2, g2, b2)


@jax.jit
def kernel(feat, edge_index, Wq, Wk, Wv, Wo, ln1_g, ln1_b,
           w1, b1, w2, b2, ln2_g, ln2_b):
    q, kv = _proj(feat, Wq, Wk, Wv)
    src = edge_index[0]
    dst = edge_index[1]
    parts, s_flat = _edge(q, kv, src, dst)
    dparts = _den(dst, s_flat)
    expander = jnp.repeat(jnp.eye(H, dtype=jnp.float32), DH, axis=1)
    return _post(parts, dparts.reshape(NW, N, H), feat, Wo, expander,
                 ln1_g.reshape(1, D), ln1_b.reshape(1, D),
                 w1, b1.reshape(1, DFF), w2, b2.reshape(1, D),
                 ln2_g.reshape(1, D), ln2_b.reshape(1, D))


# speedup vs baseline: 1.8335x; 1.1014x over previous
import jax
import jax.numpy as jnp
from jax import lax
from jax.experimental import pallas as pl
from jax.experimental.pallas import tpu as pltpu
from jax.experimental.pallas import tpu_sc as plsc

N = 10000
E = 320000
D = 128
H = 8
DH = 16
DFF = 512
CLAMP = 5.0

NC = 2
NS = 16
NW = NC * NS
E_PER_W = E // NW
CHUNK = 80
NCHUNK = E_PER_W // CHUNK
CH_A = 40
NCH_A = E_PER_W // CH_A
N_TAB = 10240
N_PER_T = N_TAB // NS


def _dot_t(x, w):
    return lax.dot_general(x, w, (((1,), (1,)), ((), ())),
                           preferred_element_type=jnp.float32)


def _proj_body(feat_ref, wq_ref, wk_ref, wv_ref, q_ref, kv_ref):
    x = feat_ref[...]
    q_ref[...] = _dot_t(x, wq_ref[...])
    kv_ref[:, :D] = _dot_t(x, wk_ref[...])
    kv_ref[:, D:] = _dot_t(x, wv_ref[...])


def _proj(feat, wq, wk, wv):
    blk = 1000
    grid = N // blk
    return pl.pallas_call(
        _proj_body,
        grid=(grid,),
        in_specs=[
            pl.BlockSpec((blk, D), lambda i: (i, 0)),
            pl.BlockSpec((D, D), lambda i: (0, 0)),
            pl.BlockSpec((D, D), lambda i: (0, 0)),
            pl.BlockSpec((D, D), lambda i: (0, 0)),
        ],
        out_specs=[
            pl.BlockSpec((blk, D), lambda i: (i, 0)),
            pl.BlockSpec((blk, 2 * D), lambda i: (i, 0)),
        ],
        out_shape=[
            jax.ShapeDtypeStruct((N, D), jnp.float32),
            jax.ShapeDtypeStruct((N, 2 * D), jnp.float32),
        ],
    )(feat, wq, wk, wv)


def _edge_sc(q_hbm, kv_hbm, src_hbm, dst_hbm, sv_hbm, s_hbm,
             sidx, didx, qbuf, kvbuf, obuf, sbuf, table, isem, gsem, ssem,
             s2sem):
    c = lax.axis_index("c")
    s = lax.axis_index("s")
    wid = c * NS + s

    lane = lax.iota(jnp.int32, 16)
    zeros = jnp.zeros((16,), jnp.float32)

    def _zb(i, _):
        for j in range(D // 16):
            obuf[0, i, pl.ds(j * 16, 16)] = zeros
        return 0
    lax.fori_loop(0, CH_A, _zb, 0)
    for k in range(N_PER_T // CH_A):
        pltpu.sync_copy(obuf.at[0],
                        table.at[pl.ds(s * N_PER_T + k * CH_A, CH_A)])
    plsc.subcore_barrier()

    base = wid * E_PER_W

    def fetch_idx(g, j):
        start = base + g * CH_A
        r = pl.multiple_of(j * 2, 2)
        pltpu.async_copy(src_hbm.at[pl.ds(start, CH_A)], sidx.at[r], isem)
        pltpu.async_copy(dst_hbm.at[pl.ds(start, CH_A)], didx.at[r], isem)

    def wait_idx(j):
        r = pl.multiple_of(j * 2, 2)
        pltpu.make_async_copy(src_hbm.at[pl.ds(0, CH_A)], sidx.at[r],
                              isem).wait()
        pltpu.make_async_copy(dst_hbm.at[pl.ds(0, CH_A)], didx.at[r],
                              isem).wait()

    def fire_gathers(j, b):
        r = pl.multiple_of(j * 2, 2)
        pltpu.async_copy(q_hbm.at[didx.at[r]], qbuf.at[b], gsem)
        pltpu.async_copy(kv_hbm.at[sidx.at[r]], kvbuf.at[b], gsem)

    def wait_gathers(j, b):
        r = pl.multiple_of(j * 2, 2)
        pltpu.make_async_copy(q_hbm.at[didx.at[r]], qbuf.at[b], gsem).wait()
        pltpu.make_async_copy(kv_hbm.at[sidx.at[r]], kvbuf.at[b], gsem).wait()

    PAIR = 2 * CH_A * H

    def fire_scatter(j, b):
        r = pl.multiple_of(j * 2, 2)
        pltpu.async_copy(obuf.at[b], table.at[didx.at[r]], ssem, add=True)

    def wait_scatter(j, b):
        r = pl.multiple_of(j * 2, 2)
        pltpu.make_async_copy(obuf.at[b], table.at[didx.at[r]], ssem).wait()

    def fire_sout(g, bp):
        start = base + (g - 1) * CH_A
        pltpu.async_copy(sbuf.at[bp, 0, pl.ds(0, PAIR)],
                         s_hbm.at[pl.ds(start * H, PAIR)], s2sem)

    def wait_sout(bp):
        pltpu.make_async_copy(sbuf.at[bp, 0, pl.ds(0, PAIR)],
                              s_hbm.at[pl.ds(0, PAIR)], s2sem).wait()

    fetch_idx(0, 0)
    fetch_idx(1, 1)
    wait_idx(0)
    fire_gathers(0, 0)

    def _chunk(g, _):
        j = g % 4
        b = g % 2
        bp = (g // 2) % 2

        @pl.when(g >= 1)
        def _():
            wait_scatter((g - 1) % 4, 1 - b)

        @pl.when(jnp.logical_and(b == 0, g >= 4))
        def _():
            wait_sout(bp)

        @pl.when(g + 2 < NCH_A)
        def _():
            fetch_idx(g + 2, (g + 2) % 4)

        wait_gathers(j, b)

        @pl.when(g + 1 < NCH_A)
        def _():
            wait_idx((g + 1) % 4)
            fire_gathers((g + 1) % 4, 1 - b)

        poff = b * CH_A * H

        def _edge_one(e, _):
            srow = zeros
            for h in range(H):
                qv = qbuf[b, e, pl.ds(h * DH, DH)]
                kv = kvbuf[b, e, pl.ds(h * DH, DH)]
                cs = plsc.cumsum(qv * kv)
                u = cs[15] * 0.25
                u = jnp.minimum(jnp.maximum(u, -CLAMP), CLAMP)
                sv = jnp.exp(jnp.full((16,), u, jnp.float32))
                vv = kvbuf[b, e, pl.ds(D + h * DH, DH)]
                obuf[b, e, pl.ds(h * DH, DH)] = sv * vv
                srow = jnp.where(lane == h, sv, srow)
            sbuf[bp, 0, pl.ds(poff + e * H, 16)] = srow
            return 0
        lax.fori_loop(0, CH_A, _edge_one, 0)

        fire_scatter(j, b)

        @pl.when(b == 1)
        def _():
            fire_sout(g, bp)
        return 0
    lax.fori_loop(0, NCH_A, _chunk, 0)

    wait_scatter((NCH_A - 1) % 4, (NCH_A - 1) % 2)
    wait_sout(0)
    wait_sout(1)
    plsc.subcore_barrier()
    pltpu.sync_copy(table.at[pl.ds(s * N_PER_T, N_PER_T)],
                    sv_hbm.at[c, pl.ds(s * N_PER_T, N_PER_T)])


def _edge(q, kv, src, dst):
    mesh = plsc.VectorSubcoreMesh(core_axis_name="c", subcore_axis_name="s",
                                  num_cores=NC, num_subcores=NS)
    f = pl.kernel(
        _edge_sc,
        out_type=(
            jax.ShapeDtypeStruct((NC, N_TAB, D), jnp.float32),
            jax.ShapeDtypeStruct((E * H,), jnp.float32),
        ),
        mesh=mesh,
        compiler_params=pltpu.CompilerParams(needs_layout_passes=False),
        scratch_types=[
            pltpu.VMEM((8, CH_A), jnp.int32),
            pltpu.VMEM((8, CH_A), jnp.int32),
            pltpu.VMEM((2, CH_A, D), jnp.float32),
            pltpu.VMEM((2, CH_A, 2 * D), jnp.float32),
            pltpu.VMEM((2, CH_A, D), jnp.float32),
            pltpu.VMEM((2, 1, 2 * CH_A * H + 64), jnp.float32),
            pltpu.VMEM_SHARED((N_TAB, D), jnp.float32),
            pltpu.SemaphoreType.DMA,
            pltpu.SemaphoreType.DMA,
            pltpu.SemaphoreType.DMA,
            pltpu.SemaphoreType.DMA,
        ],
    )
    return f(q, kv, src, dst)


def _den_sc(dst_hbm, s_hbm, dn_hbm, didx2, sbuf, dnbuf):
    c = lax.axis_index("c")
    s = lax.axis_index("s")
    wid = c * NS + s

    lane = lax.iota(jnp.int32, 16)
    lane8 = lane < H
    zeros = jnp.zeros((16,), jnp.float32)

    def _zd(i, _):
        for j in range(4):
            dnbuf[pl.ds(i * 64 + j * 16, 16)] = zeros
        return 0
    lax.fori_loop(0, N * H // 64, _zd, 0)

    base = wid * E_PER_W

    def _chunk(g, _):
        start = base + g * CHUNK
        pltpu.sync_copy(dst_hbm.at[pl.ds(start, CHUNK)],
                        didx2.at[pl.ds(0, CHUNK)])
        pltpu.sync_copy(s_hbm.at[pl.ds(start * H, CHUNK * H)],
                        sbuf.at[pl.ds(0, CHUNK * H)])

        def _edge_one(e, _):
            dvec = didx2[pl.ds(e, 16)]
            didxv = jnp.full((16,), dvec[0] * H, jnp.int32) + lane
            svec = sbuf[pl.ds(e * H, 16)]
            plsc.addupdate_scatter(dnbuf, [didxv], svec, mask=lane8)
            return 0
        lax.fori_loop(0, CHUNK, _edge_one, 0)
        return 0
    lax.fori_loop(0, NCHUNK, _chunk, 0)

    pltpu.sync_copy(dnbuf, dn_hbm.at[wid])


def _den(dst, s_flat):
    mesh = plsc.VectorSubcoreMesh(core_axis_name="c", subcore_axis_name="s",
                                  num_cores=NC, num_subcores=NS)
    f = pl.kernel(
        _den_sc,
        out_type=jax.ShapeDtypeStruct((NW, N * H), jnp.float32),
        mesh=mesh,
        compiler_params=pltpu.CompilerParams(needs_layout_passes=False),
        scratch_types=[
            pltpu.VMEM((CHUNK + 16,), jnp.int32),
            pltpu.VMEM((CHUNK * H + 16,), jnp.float32),
            pltpu.VMEM((N * H,), jnp.float32),
        ],
    )
    return f(dst, s_flat)


def _layernorm(x, g, b):
    m = jnp.mean(x, axis=-1, keepdims=True)
    xc = x - m
    v = jnp.mean(xc * xc, axis=-1, keepdims=True)
    return xc * lax.rsqrt(v + 1e-5) * g + b


def _post_body(p_ref, dn_ref, feat_ref, wo_ref, exp_ref, g1_ref, b1_ref,
               w1_ref, bb1_ref, w2_ref, bb2_ref, g2_ref, b2_ref, out_ref):
    sv = p_ref[0] + p_ref[1]
    dn = jnp.sum(dn_ref[...], axis=0)
    dn = jnp.where(dn > 0.0, dn, 1.0)
    rexp = lax.dot_general(1.0 / dn, exp_ref[...], (((1,), (0,)), ((), ())),
                           preferred_element_type=jnp.float32)
    av = sv * rexp
    uh = _dot_t(av, wo_ref[...])
    h1 = _layernorm(feat_ref[...] + uh, g1_ref[...], b1_ref[...])
    hid = jnp.maximum(_dot_t(h1, w1_ref[...]) + bb1_ref[...], 0.0)
    ffn = _dot_t(hid, w2_ref[...]) + bb2_ref[...]
    out_ref[...] = _layernorm(h1 + ffn, g2_ref[...], b2_ref[...])


def _post(parts, dparts, feat, wo, expander, g1, b1, w1, bb1, w2, bb2, g2, b2):
    blk = 1000
    grid = N // blk

    def full(shape):
        return pl.BlockSpec(shape, lambda i: tuple(0 for _ in shape))

    return pl.pallas_call(
        _post_body,
        grid=(grid,),
        in_specs=[
            pl.BlockSpec((NC, blk, D), lambda i: (0, i, 0)),
            pl.BlockSpec((NW, blk, H), lambda i: (0, i, 0)),
            pl.BlockSpec((blk, D), lambda i: (i, 0)),
            full((D, D)),
            full((H, D)),
            full((1, D)),
            full((1, D)),
            full((DFF, D)),
            full((1, DFF)),
            full((D, DFF)),
            full((1, D)),
            full((1, D)),
            full((1, D)),
        ],
        out_specs=pl.BlockSpec((blk, D), lambda i: (i, 0)),
        out_shape=jax.ShapeDtypeStruct((N, D), jnp.float32),
    )(parts, dparts, feat, wo, expander, g1, b1, w1, bb1, w2, bb2, g2, b2)


@jax.jit
def kernel(feat, edge_index, Wq, Wk, Wv, Wo, ln1_g, ln1_b,
           w1, b1, w2, b2, ln2_g, ln2_b):
    q, kv = _proj(feat, Wq, Wk, Wv)
    src = edge_index[0]
    dst = edge_index[1]
    parts, s_flat = _edge(q, kv, src, dst)
    dparts = _den(dst, s_flat)
    expander = jnp.repeat(jnp.eye(H, dtype=jnp.float32), DH, axis=1)
    return _post(parts, dparts.reshape(NW, N, H), feat, Wo, expander,
                 ln1_g.reshape(1, D), ln1_b.reshape(1, D),
                 w1, b1.reshape(1, DFF), w2, b2.reshape(1, D),
                 ln2_g.reshape(1, D), ln2_b.reshape(1, D))


# speedup vs baseline: 4.7615x; 2.5969x over previous
import jax
import jax.numpy as jnp
from jax import lax
from jax.experimental import pallas as pl
from jax.experimental.pallas import tpu as pltpu
from jax.experimental.pallas import tpu_sc as plsc

N = 10000
E = 320000
D = 128
H = 8
DH = 16
DFF = 512
CLAMP = 5.0

NC = 2
NS = 16
NW = NC * NS
E_PER_W = E // NW
CH_B = 2000
NCH_B = E_PER_W // CH_B
CH_A = 40
NCH_A = E_PER_W // CH_A
N_TAB = 10240
N_PER_T = N_TAB // NS


def _dot_t(x, w):
    return lax.dot_general(x, w, (((1,), (1,)), ((), ())),
                           preferred_element_type=jnp.float32)


def _proj_body(feat_ref, wq_ref, wk_ref, wv_ref, q_ref, kv_ref):
    x = feat_ref[...]
    q_ref[...] = _dot_t(x, wq_ref[...])
    kv_ref[:, :D] = _dot_t(x, wk_ref[...])
    kv_ref[:, D:] = _dot_t(x, wv_ref[...])


def _proj(feat, wq, wk, wv):
    blk = 1000
    grid = N // blk
    return pl.pallas_call(
        _proj_body,
        grid=(grid,),
        in_specs=[
            pl.BlockSpec((blk, D), lambda i: (i, 0)),
            pl.BlockSpec((D, D), lambda i: (0, 0)),
            pl.BlockSpec((D, D), lambda i: (0, 0)),
            pl.BlockSpec((D, D), lambda i: (0, 0)),
        ],
        out_specs=[
            pl.BlockSpec((blk, D), lambda i: (i, 0)),
            pl.BlockSpec((blk, 2 * D), lambda i: (i, 0)),
        ],
        out_shape=[
            jax.ShapeDtypeStruct((N, D), jnp.float32),
            jax.ShapeDtypeStruct((N, 2 * D), jnp.float32),
        ],
    )(feat, wq, wk, wv)


def _edge_sc(q_hbm, kv_hbm, src_hbm, dst_hbm, sv_hbm, s_hbm,
             sidx, didx, qbuf, kvbuf, obuf, sbuf, table, isem, gsem, ssem,
             s2sem):
    c = lax.axis_index("c")
    s = lax.axis_index("s")
    wid = c * NS + s

    lane = lax.iota(jnp.int32, 16)
    zeros = jnp.zeros((16,), jnp.float32)

    ix8 = lane ^ 8
    ix4 = lane ^ 4
    ix2 = lane ^ 2
    ix1 = lane ^ 1
    p1v = (lane + 4) & 15
    p2v = (lane - 8) & 15
    idx4v = 4 * ((lane & 7) >> 1) + (lane & 1)
    idx5v = (2 * lane) & 15
    lt4 = lane < 4
    lt8 = lane < 8
    lt12 = lane < 12

    gdn = lax.GatherDimensionNumbers(offset_dims=(), collapsed_slice_dims=(0,),
                                     start_index_map=(0,))

    def _perm(x, idx):
        return lax.gather(x, idx[:, None], gdn, (1,),
                          mode=lax.GatherScatterMode.PROMISE_IN_BOUNDS)

    def _zb(i, _):
        for j in range(D // 16):
            obuf[0, i, pl.ds(j * 16, 16)] = zeros
        return 0
    lax.fori_loop(0, CH_A, _zb, 0)
    for k in range(N_PER_T // CH_A):
        pltpu.sync_copy(obuf.at[0],
                        table.at[pl.ds(s * N_PER_T + k * CH_A, CH_A)])
    plsc.subcore_barrier()

    base = wid * E_PER_W

    def fetch_idx(g, j):
        start = base + g * CH_A
        r = pl.multiple_of(j * 2, 2)
        pltpu.async_copy(src_hbm.at[pl.ds(start, CH_A)], sidx.at[r], isem)
        pltpu.async_copy(dst_hbm.at[pl.ds(start, CH_A)], didx.at[r], isem)

    def wait_idx(j):
        r = pl.multiple_of(j * 2, 2)
        pltpu.make_async_copy(src_hbm.at[pl.ds(0, CH_A)], sidx.at[r],
                              isem).wait()
        pltpu.make_async_copy(dst_hbm.at[pl.ds(0, CH_A)], didx.at[r],
                              isem).wait()

    def fire_gathers(j, b):
        r = pl.multiple_of(j * 2, 2)
        pltpu.async_copy(q_hbm.at[didx.at[r]], qbuf.at[b], gsem)
        pltpu.async_copy(kv_hbm.at[sidx.at[r]], kvbuf.at[b], gsem)

    def wait_gathers(j, b):
        r = pl.multiple_of(j * 2, 2)
        pltpu.make_async_copy(q_hbm.at[didx.at[r]], qbuf.at[b], gsem).wait()
        pltpu.make_async_copy(kv_hbm.at[sidx.at[r]], kvbuf.at[b], gsem).wait()

    PAIR = 2 * CH_A * H

    def fire_scatter(j, b):
        r = pl.multiple_of(j * 2, 2)
        pltpu.async_copy(obuf.at[b], table.at[didx.at[r]], ssem, add=True)

    def wait_scatter(j, b):
        r = pl.multiple_of(j * 2, 2)
        pltpu.make_async_copy(obuf.at[b], table.at[didx.at[r]], ssem).wait()

    def fire_sout(g, bp):
        start = base + (g - 1) * CH_A
        pltpu.async_copy(sbuf.at[bp, 0, pl.ds(0, PAIR)],
                         s_hbm.at[pl.ds(start * H, PAIR)], s2sem)

    def wait_sout(bp):
        pltpu.make_async_copy(sbuf.at[bp, 0, pl.ds(0, PAIR)],
                              s_hbm.at[pl.ds(0, PAIR)], s2sem).wait()

    fetch_idx(0, 0)
    fetch_idx(1, 1)
    wait_idx(0)
    fire_gathers(0, 0)

    def _chunk(g, _):
        j = g % 4
        b = g % 2
        bp = (g // 2) % 2

        @pl.when(g >= 1)
        def _():
            wait_scatter((g - 1) % 4, 1 - b)

        @pl.when(jnp.logical_and(b == 0, g >= 4))
        def _():
            wait_sout(bp)

        @pl.when(g + 2 < NCH_A)
        def _():
            fetch_idx(g + 2, (g + 2) % 4)

        wait_gathers(j, b)

        @pl.when(g + 1 < NCH_A)
        def _():
            wait_idx((g + 1) % 4)
            fire_gathers((g + 1) % 4, 1 - b)

        poff = b * CH_A * H

        def _edge_one(e4, _):
            for ee in range(4):
                e = e4 * 4 + ee
                ps = [qbuf[b, e, pl.ds(h * DH, DH)] *
                      kvbuf[b, e, pl.ds(h * DH, DH)] for h in range(H)]
                aa = [ps[i] + _perm(ps[i], ix8) for i in range(8)]
                mm = [jnp.where(lt8, aa[2 * i], aa[2 * i + 1])
                      for i in range(4)]
                bb = [mm[i] + _perm(mm[i], ix4) for i in range(4)]
                cc = [jnp.where(lt4, bb[2 * j],
                                jnp.where(lt8, _perm(bb[2 * j], p1v),
                                          jnp.where(lt12,
                                                    _perm(bb[2 * j + 1], p2v),
                                                    bb[2 * j + 1])))
                      for j in range(2)]
                dd = [cc[j] + _perm(cc[j], ix2) for j in range(2)]
                s2 = jnp.where(lt8, _perm(dd[0], idx4v), _perm(dd[1], idx4v))
                sp = s2 + _perm(s2, ix1)
                u = _perm(sp, idx5v) * 0.25
                u = jnp.minimum(jnp.maximum(u, -CLAMP), CLAMP)
                svec = jnp.exp(u)
                sbuf[bp, 0, pl.ds(poff + e * H, 16)] = svec
                for h in range(H):
                    sh = _perm(svec, lane * 0 + h)
                    vv = kvbuf[b, e, pl.ds(D + h * DH, DH)]
                    obuf[b, e, pl.ds(h * DH, DH)] = sh * vv
            return 0
        lax.fori_loop(0, CH_A // 4, _edge_one, 0)

        fire_scatter(j, b)

        @pl.when(b == 1)
        def _():
            fire_sout(g, bp)
        return 0
    lax.fori_loop(0, NCH_A, _chunk, 0)

    wait_scatter((NCH_A - 1) % 4, (NCH_A - 1) % 2)
    wait_sout(0)
    wait_sout(1)
    plsc.subcore_barrier()
    pltpu.sync_copy(table.at[pl.ds(s * N_PER_T, N_PER_T)],
                    sv_hbm.at[c, pl.ds(s * N_PER_T, N_PER_T)])


def _edge(q, kv, src, dst):
    mesh = plsc.VectorSubcoreMesh(core_axis_name="c", subcore_axis_name="s",
                                  num_cores=NC, num_subcores=NS)
    f = pl.kernel(
        _edge_sc,
        out_type=(
            jax.ShapeDtypeStruct((NC, N_TAB, D), jnp.float32),
            jax.ShapeDtypeStruct((E * H,), jnp.float32),
        ),
        mesh=mesh,
        compiler_params=pltpu.CompilerParams(needs_layout_passes=False),
        scratch_types=[
            pltpu.VMEM((8, CH_A), jnp.int32),
            pltpu.VMEM((8, CH_A), jnp.int32),
            pltpu.VMEM((2, CH_A, D), jnp.float32),
            pltpu.VMEM((2, CH_A, 2 * D), jnp.float32),
            pltpu.VMEM((2, CH_A, D), jnp.float32),
            pltpu.VMEM((2, 1, 2 * CH_A * H + 64), jnp.float32),
            pltpu.VMEM_SHARED((N_TAB, D), jnp.float32),
            pltpu.SemaphoreType.DMA,
            pltpu.SemaphoreType.DMA,
            pltpu.SemaphoreType.DMA,
            pltpu.SemaphoreType.DMA,
        ],
    )
    return f(q, kv, src, dst)


def _den_sc(dst_hbm, s_hbm, dn_hbm, didx2, sbuf, dnbuf):
    c = lax.axis_index("c")
    s = lax.axis_index("s")
    wid = c * NS + s

    lane = lax.iota(jnp.int32, 16)
    lane8 = lane < H
    zeros = jnp.zeros((16,), jnp.float32)

    def _zd(i, _):
        for j in range(4):
            dnbuf[pl.ds(i * 64 + j * 16, 16)] = zeros
        return 0
    lax.fori_loop(0, N * H // 64, _zd, 0)

    base = wid * E_PER_W

    def _chunk(g, _):
        start = base + g * CH_B
        pltpu.sync_copy(dst_hbm.at[pl.ds(start, CH_B)],
                        didx2.at[pl.ds(0, CH_B)])
        pltpu.sync_copy(s_hbm.at[pl.ds(start * H, CH_B * H)],
                        sbuf.at[pl.ds(0, CH_B * H)])

        def _edge_one(e, _):
            dvec = didx2[pl.ds(e, 16)]
            didxv = jnp.full((16,), dvec[0] * H, jnp.int32) + lane
            svec = sbuf[pl.ds(e * H, 16)]
            plsc.addupdate_scatter(dnbuf, [didxv], svec, mask=lane8)
            return 0
        lax.fori_loop(0, CH_B, _edge_one, 0)
        return 0
    lax.fori_loop(0, NCH_B, _chunk, 0)

    pltpu.sync_copy(dnbuf, dn_hbm.at[wid])


def _den(dst, s_flat):
    mesh = plsc.VectorSubcoreMesh(core_axis_name="c", subcore_axis_name="s",
                                  num_cores=NC, num_subcores=NS)
    f = pl.kernel(
        _den_sc,
        out_type=jax.ShapeDtypeStruct((NW, N * H), jnp.float32),
        mesh=mesh,
        compiler_params=pltpu.CompilerParams(needs_layout_passes=False),
        scratch_types=[
            pltpu.VMEM((CH_B + 16,), jnp.int32),
            pltpu.VMEM((CH_B * H + 16,), jnp.float32),
            pltpu.VMEM((N * H,), jnp.float32),
        ],
    )
    return f(dst, s_flat)


def _layernorm(x, g, b):
    m = jnp.mean(x, axis=-1, keepdims=True)
    xc = x - m
    v = jnp.mean(xc * xc, axis=-1, keepdims=True)
    return xc * lax.rsqrt(v + 1e-5) * g + b


def _post_body(p_ref, dn_ref, feat_ref, wo_ref, exp_ref, g1_ref, b1_ref,
               w1_ref, bb1_ref, w2_ref, bb2_ref, g2_ref, b2_ref, out_ref):
    sv = p_ref[0] + p_ref[1]
    dn = jnp.sum(dn_ref[...], axis=0)
    dn = jnp.where(dn > 0.0, dn, 1.0)
    rexp = lax.dot_general(1.0 / dn, exp_ref[...], (((1,), (0,)), ((), ())),
                           preferred_element_type=jnp.float32)
    av = sv * rexp
    uh = _dot_t(av, wo_ref[...])
    h1 = _layernorm(feat_ref[...] + uh, g1_ref[...], b1_ref[...])
    hid = jnp.maximum(_dot_t(h1, w1_ref[...]) + bb1_ref[...], 0.0)
    ffn = _dot_t(hid, w2_ref[...]) + bb2_ref[...]
    out_ref[...] = _layernorm(h1 + ffn, g2_ref[...], b2_ref[...])


def _post(parts, dparts, feat, wo, expander, g1, b1, w1, bb1, w2, bb2, g2, b2):
    blk = 1000
    grid = N // blk

    def full(shape):
        return pl.BlockSpec(shape, lambda i: tuple(0 for _ in shape))

    return pl.pallas_call(
        _post_body,
        grid=(grid,),
        in_specs=[
            pl.BlockSpec((NC, blk, D), lambda i: (0, i, 0)),
            pl.BlockSpec((NW, blk, H), lambda i: (0, i, 0)),
            pl.BlockSpec((blk, D), lambda i: (i, 0)),
            full((D, D)),
            full((H, D)),
            full((1, D)),
            full((1, D)),
            full((DFF, D)),
            full((1, DFF)),
            full((D, DFF)),
            full((1, D)),
            full((1, D)),
            full((1, D)),
        ],
        out_specs=pl.BlockSpec((blk, D), lambda i: (i, 0)),
        out_shape=jax.ShapeDtypeStruct((N, D), jnp.float32),
    )(parts, dparts, feat, wo, expander, g1, b1, w1, bb1, w2, bb2, g2, b2)


@jax.jit
def kernel(feat, edge_index, Wq, Wk, Wv, Wo, ln1_g, ln1_b,
           w1, b1, w2, b2, ln2_g, ln2_b):
    q, kv = _proj(feat, Wq, Wk, Wv)
    src = edge_index[0]
    dst = edge_index[1]
    parts, s_flat = _edge(q, kv, src, dst)
    dparts = _den(dst, s_flat)
    expander = jnp.repeat(jnp.eye(H, dtype=jnp.float32), DH, axis=1)
    return _post(parts, dparts.reshape(NW, N, H), feat, Wo, expander,
                 ln1_g.reshape(1, D), ln1_b.reshape(1, D),
                 w1, b1.reshape(1, DFF), w2, b2.reshape(1, D),
                 ln2_g.reshape(1, D), ln2_b.reshape(1, D))
